# baseline (speedup 1.0000x reference)
import functools

import jax
import jax.numpy as jnp
from jax import lax
from jax.experimental import pallas as pl
from jax.experimental.pallas import tpu as pltpu
from jax.experimental.pallas import tpu_sc as plsc

B = 16384
VOCAB = 1000000
S = 200
E = 64
NC = 2
NS = 16
NW = NC * NS
BPW = B // NW
H = S // 2
CH = 8
HPC = CH * 2
NCHUNK = BPW // CH
NB = NCHUNK // 2
RING = 8
NLANE = 4

_mesh = plsc.VectorSubcoreMesh(core_axis_name="c", subcore_axis_name="s")


NJF = (VOCAB // 128)
NJ_HI = NJF // NW + 1


@functools.partial(
    pl.kernel,
    out_type=jax.ShapeDtypeStruct((VOCAB // 2, 128), jnp.float32),
    mesh=_mesh,
    compiler_params=pltpu.CompilerParams(use_tc_tiling_on_sc=True,
                                        needs_layout_passes=False),
    scratch_types=[
        pltpu.VMEM((2, E, 128), jnp.float32),
        pltpu.VMEM((2, E, 128), jnp.float32),
        pltpu.VMEM((E, 64), jnp.float32),
        pltpu.SemaphoreType.DMA((2,)),
        pltpu.SemaphoreType.DMA((2,)),
    ],
)
def _repack(tt_hbm, out_hbm, in_v, tr_v, tail_v, isem, osem):
    wid = lax.axis_index("s") * NC + lax.axis_index("c")
    nj = jnp.where(wid < NJF % NW, NJ_HI, NJ_HI - 1)
    iota = lax.iota(jnp.int32, 16)
    half = lax.shift_right_logical(iota, 1)
    par64 = (iota & 1) * 64
    idx_rows = [cb * 8 + half for cb in range(8)]

    def in_copy(t, buf):
        j = wid + NW * t
        return pltpu.make_async_copy(
            tt_hbm.at[:, pl.ds(j * 128, 128)], in_v.at[buf], isem.at[buf])

    def out_copy(t, buf):
        j = wid + NW * t
        return pltpu.make_async_copy(
            tr_v.at[buf], out_hbm.at[pl.ds(j * 64, E)], osem.at[buf])

    def transpose64(src_ref, buf, ncb):
        def ebody(e, c):
            for cb in range(ncb):
                row = src_ref[e, pl.ds(cb * 16, 16)]
                plsc.store_scatter(tr_v.at[buf], [idx_rows[cb], par64 + e],
                                   row)
            return c
        lax.fori_loop(0, E, ebody, 0)

    in_copy(0, 0).start()

    def body(t2, carry):
        for par in range(2):
            t = 2 * t2 + par

            @pl.when(t < nj)
            def _():
                @pl.when(t + 1 < nj)
                def _():
                    in_copy(t + 1, 1 - par).start()
                in_copy(t, par).wait()

                @pl.when(t >= 2)
                def _():
                    out_copy(0, par).wait()
                transpose64(in_v.at[par], par, 8)
                out_copy(t, par).start()
        return carry

    lax.fori_loop(0, (NJ_HI + 1) // 2, body, 0)
    out_copy(0, 0).wait()
    out_copy(0, 1).wait()

    @pl.when(wid == 0)
    def _():
        pltpu.sync_copy(tt_hbm.at[:, pl.ds(NJF * 128, 64)], tail_v)
        transpose64(tail_v, 0, 4)
        pltpu.sync_copy(tr_v.at[0, pl.ds(0, 32)],
                        out_hbm.at[pl.ds(NJF * 64, 32)])


@functools.partial(
    pl.kernel,
    out_type=jax.ShapeDtypeStruct((B, E), jnp.float32),
    mesh=_mesh,
    compiler_params=pltpu.CompilerParams(use_tc_tiling_on_sc=False),
    scratch_types=[
        pltpu.VMEM((2, HPC, H), jnp.int32),
        pltpu.VMEM((RING, H, E), jnp.float32),
        pltpu.VMEM((2, CH, E), jnp.float32),
        pltpu.SemaphoreType.DMA((RING,)),
        pltpu.SemaphoreType.DMA((2,)),
        pltpu.SemaphoreType.DMA((2,)),
    ],
)
def _pooled_lookup(x_hbm, table_hbm, out_hbm, idx_v, rows_v, out_v,
                   gsem, isem, osem):
    wid = lax.axis_index("s") * NC + lax.axis_index("c")
    wbase_s = wid * BPW
    wbase_h = wbase_s * 2

    def idx_copy(chunk, buf):
        return pltpu.make_async_copy(
            x_hbm.at[pl.ds(wbase_h + chunk * HPC, HPC)],
            idx_v.at[buf], isem.at[buf])

    def gather(ibuf, irow, slot):
        return pltpu.make_async_copy(
            table_hbm.at[idx_v.at[ibuf, irow]], rows_v.at[slot],
            gsem.at[slot])

    def out_copy(chunk, buf):
        return pltpu.make_async_copy(
            out_v.at[buf], out_hbm.at[pl.ds(wbase_s + chunk * CH, CH)],
            osem.at[buf])

    idx_copy(0, 0).start()
    idx_copy(0, 0).wait()
    for k in range(RING):
        gather(0, k, k).start()

    def body(ci2, carry):
        not_last = ci2 < NB - 1
        acc = tuple(jnp.zeros((16,), jnp.float32) for _ in range(2 * NLANE))
        for hp in range(2 * HPC):
            slot = hp % RING
            pc = hp // HPC

            if hp == 0:
                idx_copy(2 * ci2 + 1, 1).start()

                @pl.when(ci2 > 0)
                def _():
                    out_copy(0, 0).wait()
            if hp == RING:
                idx_copy(0, 1).wait()
            if hp == HPC:
                @pl.when(ci2 > 0)
                def _():
                    out_copy(0, 1).wait()

                @pl.when(not_last)
                def _():
                    idx_copy(2 * ci2 + 2, 0).start()
            if hp == 2 * HPC - RING:
                @pl.when(not_last)
                def _():
                    idx_copy(0, 0).wait()

            gather(pc, hp % HPC, slot).wait()

            def red(i, a, _slot=slot):
                a = list(a)
                r = i * 4
                for rr in range(4):
                    p = rr % 2
                    for c in range(NLANE):
                        a[c * 2 + p] = a[c * 2 + p] + rows_v[
                            _slot, r + rr, pl.ds(c * 16, 16)]
                return tuple(a)

            acc = lax.fori_loop(0, H // 4, red, acc)

            h2 = hp + RING
            if h2 < 2 * HPC:
                gather(h2 // HPC, h2 % HPC, slot).start()
            else:
                @pl.when(not_last)
                def _():
                    gather(0, h2 - 2 * HPC, slot).start()

            if hp % 2 == 1:
                sp = (hp // 2) % CH
                for c in range(NLANE):
                    out_v[pc, sp, pl.ds(c * 16, 16)] = (
                        acc[c * 2] + acc[c * 2 + 1]) * (1.0 / S)
                acc = tuple(jnp.zeros((16,), jnp.float32)
                            for _ in range(2 * NLANE))
            if hp == HPC - 1:
                out_copy(2 * ci2, 0).start()
            if hp == 2 * HPC - 1:
                out_copy(2 * ci2 + 1, 1).start()
        return carry

    lax.fori_loop(0, NB, body, 0)
    out_copy(0, 0).wait()
    out_copy(0, 1).wait()


def kernel(x, x_len, table):
    del x_len
    x_halves = x.reshape(B * 2, H)
    tpack = _repack(table.T)
    tlin = tpack.reshape(VOCAB, E)
    return _pooled_lookup(x_halves, tlin)


# speedup vs baseline: 1.8488x; 1.8488x over previous
import functools

import jax
import jax.numpy as jnp
from jax import lax
from jax.experimental import pallas as pl
from jax.experimental.pallas import tpu as pltpu
from jax.experimental.pallas import tpu_sc as plsc

B = 16384
VOCAB = 1000000
S = 200
E = 64
NC = 2
NS = 16
NW = NC * NS
BPW = B // NW
H = S // 2
CH = 8
HPC = CH * 2
NCHUNK = BPW // CH
NB = NCHUNK // 2
RING = 8
NLANE = 4

_mesh = plsc.VectorSubcoreMesh(core_axis_name="c", subcore_axis_name="s")


NJF = (VOCAB // 128)
NJ_HI = NJF // NW + 1


@functools.partial(
    pl.kernel,
    out_type=jax.ShapeDtypeStruct((VOCAB // 2, 128), jnp.float32),
    mesh=_mesh,
    compiler_params=pltpu.CompilerParams(use_tc_tiling_on_sc=True,
                                        needs_layout_passes=False),
    scratch_types=[
        pltpu.VMEM((2, E, 128), jnp.float32),
        pltpu.VMEM((2, E, 128), jnp.float32),
        pltpu.VMEM((E, 64), jnp.float32),
        pltpu.SemaphoreType.DMA((2,)),
        pltpu.SemaphoreType.DMA((2,)),
    ],
)
def _repack(tt_hbm, out_hbm, in_v, tr_v, tail_v, isem, osem):
    wid = lax.axis_index("s") * NC + lax.axis_index("c")
    nj = jnp.where(wid < NJF % NW, NJ_HI, NJ_HI - 1)
    iota = lax.iota(jnp.int32, 16)
    rot = [(iota + d) & 15 for d in range(16)]
    rot_q = [lax.shift_right_logical(r, 1) for r in rot]
    rot_pe = [((r & 1) << 6) + iota for r in rot]

    def in_copy(t, buf):
        j = wid + NW * t
        return pltpu.make_async_copy(
            tt_hbm.at[:, pl.ds(j * 128, 128)], in_v.at[buf], isem.at[buf])

    def out_copy(t, buf):
        j = wid + NW * t
        return pltpu.make_async_copy(
            tr_v.at[buf], out_hbm.at[pl.ds(j * 64, E)], osem.at[buf])

    def transpose64(src_ref, buf, ncb):
        def kbody(k, c):
            e0 = (k & 3) << 4
            c0 = lax.shift_right_logical(k, 2) << 4
            c0h = lax.shift_right_logical(c0, 1)
            lrow = iota + e0
            for d in range(16):
                v = plsc.load_gather(src_ref, [lrow, rot[d] + c0])
                plsc.store_scatter(tr_v.at[buf],
                                   [rot_q[d] + c0h, rot_pe[d] + e0], v)
            return c
        lax.fori_loop(0, ncb * 4, kbody, 0)

    in_copy(0, 0).start()

    def body(t2, carry):
        for par in range(2):
            t = 2 * t2 + par

            @pl.when(t < nj)
            def _():
                @pl.when(t + 1 < nj)
                def _():
                    in_copy(t + 1, 1 - par).start()
                in_copy(t, par).wait()

                @pl.when(t >= 2)
                def _():
                    out_copy(0, par).wait()
                transpose64(in_v.at[par], par, 8)
                out_copy(t, par).start()
        return carry

    lax.fori_loop(0, (NJ_HI + 1) // 2, body, 0)
    out_copy(0, 0).wait()
    out_copy(0, 1).wait()

    @pl.when(wid == 0)
    def _():
        pltpu.sync_copy(tt_hbm.at[:, pl.ds(NJF * 128, 64)], tail_v)
        transpose64(tail_v, 0, 4)
        pltpu.sync_copy(tr_v.at[0, pl.ds(0, 32)],
                        out_hbm.at[pl.ds(NJF * 64, 32)])


@functools.partial(
    pl.kernel,
    out_type=jax.ShapeDtypeStruct((B, E), jnp.float32),
    mesh=_mesh,
    compiler_params=pltpu.CompilerParams(use_tc_tiling_on_sc=False),
    scratch_types=[
        pltpu.VMEM((2, HPC, H), jnp.int32),
        pltpu.VMEM((RING, H, E), jnp.float32),
        pltpu.VMEM((2, CH, E), jnp.float32),
        pltpu.SemaphoreType.DMA((RING,)),
        pltpu.SemaphoreType.DMA((2,)),
        pltpu.SemaphoreType.DMA((2,)),
    ],
)
def _pooled_lookup(x_hbm, table_hbm, out_hbm, idx_v, rows_v, out_v,
                   gsem, isem, osem):
    wid = lax.axis_index("s") * NC + lax.axis_index("c")
    wbase_s = wid * BPW
    wbase_h = wbase_s * 2

    def idx_copy(chunk, buf):
        return pltpu.make_async_copy(
            x_hbm.at[pl.ds(wbase_h + chunk * HPC, HPC)],
            idx_v.at[buf], isem.at[buf])

    def gather(ibuf, irow, slot):
        return pltpu.make_async_copy(
            table_hbm.at[idx_v.at[ibuf, irow]], rows_v.at[slot],
            gsem.at[slot])

    def out_copy(chunk, buf):
        return pltpu.make_async_copy(
            out_v.at[buf], out_hbm.at[pl.ds(wbase_s + chunk * CH, CH)],
            osem.at[buf])

    idx_copy(0, 0).start()
    idx_copy(0, 0).wait()
    for k in range(RING):
        gather(0, k, k).start()

    def body(ci2, carry):
        not_last = ci2 < NB - 1
        acc = tuple(jnp.zeros((16,), jnp.float32) for _ in range(2 * NLANE))
        for hp in range(2 * HPC):
            slot = hp % RING
            pc = hp // HPC

            if hp == 0:
                idx_copy(2 * ci2 + 1, 1).start()

                @pl.when(ci2 > 0)
                def _():
                    out_copy(0, 0).wait()
            if hp == RING:
                idx_copy(0, 1).wait()
            if hp == HPC:
                @pl.when(ci2 > 0)
                def _():
                    out_copy(0, 1).wait()

                @pl.when(not_last)
                def _():
                    idx_copy(2 * ci2 + 2, 0).start()
            if hp == 2 * HPC - RING:
                @pl.when(not_last)
                def _():
                    idx_copy(0, 0).wait()

            gather(pc, hp % HPC, slot).wait()

            def red(i, a, _slot=slot):
                a = list(a)
                r = i * 4
                for rr in range(4):
                    p = rr % 2
                    for c in range(NLANE):
                        a[c * 2 + p] = a[c * 2 + p] + rows_v[
                            _slot, r + rr, pl.ds(c * 16, 16)]
                return tuple(a)

            acc = lax.fori_loop(0, H // 4, red, acc)

            h2 = hp + RING
            if h2 < 2 * HPC:
                gather(h2 // HPC, h2 % HPC, slot).start()
            else:
                @pl.when(not_last)
                def _():
                    gather(0, h2 - 2 * HPC, slot).start()

            if hp % 2 == 1:
                sp = (hp // 2) % CH
                for c in range(NLANE):
                    out_v[pc, sp, pl.ds(c * 16, 16)] = (
                        acc[c * 2] + acc[c * 2 + 1]) * (1.0 / S)
                acc = tuple(jnp.zeros((16,), jnp.float32)
                            for _ in range(2 * NLANE))
            if hp == HPC - 1:
                out_copy(2 * ci2, 0).start()
            if hp == 2 * HPC - 1:
                out_copy(2 * ci2 + 1, 1).start()
        return carry

    lax.fori_loop(0, NB, body, 0)
    out_copy(0, 0).wait()
    out_copy(0, 1).wait()


def kernel(x, x_len, table):
    del x_len
    x_halves = x.reshape(B * 2, H)
    tpack = _repack(table.T)
    tlin = tpack.reshape(VOCAB, E)
    return _pooled_lookup(x_halves, tlin)
